# baseline (speedup 1.0000x reference)
import jax
import jax.numpy as jnp
from jax.experimental import pallas as pl

N = 10000
D = 64
BM = 400


def _matmul_block(a_ref, e_ref, o_ref):
    a16 = a_ref[...].astype(jnp.bfloat16)
    e16 = e_ref[...].astype(jnp.bfloat16)
    o_ref[...] = jnp.dot(a16, e16, preferred_element_type=jnp.float32)


def kernel(A_hat, E):
    return pl.pallas_call(
        _matmul_block,
        grid=(N // BM,),
        in_specs=[
            pl.BlockSpec((BM, N), lambda i: (i, 0)),
            pl.BlockSpec((N, D), lambda i: (0, 0)),
        ],
        out_specs=pl.BlockSpec((BM, D), lambda i: (i, 0)),
        out_shape=jax.ShapeDtypeStruct((N, D), jnp.float32),
    )(A_hat, E)
